# baseline (speedup 1.0000x reference)
import functools
import math

import jax
import jax.numpy as jnp
import numpy as np
from jax import lax
from jax.experimental import pallas as pl
from jax.experimental.pallas import tpu as pltpu
from jax.experimental.pallas import tpu_sc as plsc

N = 10000
E = 320000
D = 128
EDIM = 16
NH = 4
CH = 32
HC = NH * CH

NC, NS = 2, 16
NWORK = NC * NS
EPW = E // NWORK
BE = 16
NPH = 5
PHE = EPW // NPH
PCH = PHE // BE
NPAD = 10240
RPS = NPAD // NS
DPK = NPAD // 8
DPS = DPK // NS
ZROWS = 16
INV_SQRT_C = 1.0 / math.sqrt(CH)

NBLK = 5
BN = N // NBLK
EBLK = 64
BEP = E // EBLK
BD = DPK // NBLK

_G = np.zeros((HC, 8 * HC), np.float32)
for _p in range(8):
    for _h in range(NH):
        _G[16 * _p + _h, 128 * _p + 32 * _h:128 * _p + 32 * (_h + 1)] = 1.0


def _dot(a, b):
    return lax.dot_general(a, b, (((1,), (0,)), ((), ())),
                           preferred_element_type=jnp.float32,
                           precision=lax.Precision.HIGHEST)


def _proj_body(x_ref, w_ref, b_ref, oq, ok, ov, os_):
    acc = _dot(x_ref[...], w_ref[...]) + b_ref[...]
    oq[...] = acc[:, 0 * HC:1 * HC]
    ok[...] = acc[:, 1 * HC:2 * HC]
    ov[...] = acc[:, 2 * HC:3 * HC]
    os_[...] = acc[:, 3 * HC:4 * HC]


def _node_proj(x, w, b):
    return pl.pallas_call(
        _proj_body,
        grid=(NBLK,),
        in_specs=[
            pl.BlockSpec((BN, D), lambda i: (i, 0)),
            pl.BlockSpec((D, 4 * HC), lambda i: (0, 0)),
            pl.BlockSpec((1, 4 * HC), lambda i: (0, 0)),
        ],
        out_specs=[pl.BlockSpec((BN, HC), lambda i: (i, 0))] * 4,
        out_shape=[jax.ShapeDtypeStruct((N, HC), jnp.float32)] * 4,
    )(x, w, b)


def _edge_proj_body(ea_ref, w_ref, o1, o2):
    acc = _dot(ea_ref[...], w_ref[...])
    o1[...] = acc[:, 0:HC]
    o2[...] = acc[:, HC:2 * HC]


def _edge_proj(ea, wee):
    return pl.pallas_call(
        _edge_proj_body,
        grid=(EBLK,),
        in_specs=[
            pl.BlockSpec((BEP, EDIM), lambda i: (i, 0)),
            pl.BlockSpec((EDIM, 2 * HC), lambda i: (0, 0)),
        ],
        out_specs=[pl.BlockSpec((BEP, HC), lambda i: (i, 0))] * 2,
        out_shape=[jax.ShapeDtypeStruct((E, HC), jnp.float32)] * 2,
    )(ea, wee)


def _den_proj_body(d0_ref, d1_ref, g_ref, o_ref):
    o_ref[...] = _dot(d0_ref[0] + d1_ref[0], g_ref[...])


def _den_proj(ad, g):
    return pl.pallas_call(
        _den_proj_body,
        grid=(NBLK,),
        in_specs=[
            pl.BlockSpec((1, BD, HC), lambda i: (0, i, 0)),
            pl.BlockSpec((1, BD, HC), lambda i: (1, i, 0)),
            pl.BlockSpec((HC, 8 * HC), lambda i: (0, 0)),
        ],
        out_specs=pl.BlockSpec((BD, 8 * HC), lambda i: (i, 0)),
        out_shape=jax.ShapeDtypeStruct((DPK, 8 * HC), jnp.float32),
    )(ad, ad, g)


def _layer2_body(v0, v1, db, s1_ref, w_ref, b_ref, oq, ok, ov, os_):
    h = jnp.maximum(
        (v0[0] + v1[0]) / (db[...] + 1e-30) + s1_ref[...], 0.0)
    acc = _dot(h, w_ref[...]) + b_ref[...]
    oq[...] = acc[:, 0 * HC:1 * HC]
    ok[...] = acc[:, 1 * HC:2 * HC]
    ov[...] = acc[:, 2 * HC:3 * HC]
    os_[...] = acc[:, 3 * HC:4 * HC]


def _layer2_proj(av, db, s1, w, b):
    return pl.pallas_call(
        _layer2_body,
        grid=(NBLK,),
        in_specs=[
            pl.BlockSpec((1, BN, HC), lambda i: (0, i, 0)),
            pl.BlockSpec((1, BN, HC), lambda i: (1, i, 0)),
            pl.BlockSpec((BN, HC), lambda i: (i, 0)),
            pl.BlockSpec((BN, HC), lambda i: (i, 0)),
            pl.BlockSpec((D, 4 * HC), lambda i: (0, 0)),
            pl.BlockSpec((1, 4 * HC), lambda i: (0, 0)),
        ],
        out_specs=[pl.BlockSpec((BN, HC), lambda i: (i, 0))] * 4,
        out_shape=[jax.ShapeDtypeStruct((N, HC), jnp.float32)] * 4,
    )(av, av, db, s1, w, b)


def _final_body(v0, v1, db, s2_ref, o_ref):
    o_ref[...] = (v0[0] + v1[0]) / (db[...] + 1e-30) + s2_ref[...]


def _final(av, db, s2):
    return pl.pallas_call(
        _final_body,
        grid=(NBLK,),
        in_specs=[
            pl.BlockSpec((1, BN, HC), lambda i: (0, i, 0)),
            pl.BlockSpec((1, BN, HC), lambda i: (1, i, 0)),
            pl.BlockSpec((BN, HC), lambda i: (i, 0)),
            pl.BlockSpec((BN, HC), lambda i: (i, 0)),
        ],
        out_specs=pl.BlockSpec((BN, HC), lambda i: (i, 0)),
        out_shape=jax.ShapeDtypeStruct((N, HC), jnp.float32),
    )(av, av, db, s2)


_MESH = plsc.VectorSubcoreMesh(core_axis_name="c", subcore_axis_name="s",
                               num_cores=NC, num_subcores=NS)

_GDN = lax.GatherDimensionNumbers(offset_dims=(), collapsed_slice_dims=(0,),
                                  start_index_map=(0,))


def _permute(v, idx):
    return lax.gather(v, idx[:, None], _GDN, (1,), indices_are_sorted=False,
                      unique_indices=False,
                      mode=lax.GatherScatterMode.PROMISE_IN_BOUNDS)


def _lane_sum_splat(v, lane):
    for sft in (8, 4, 2, 1):
        v = v + _permute(v, (lane & (sft - 1)) + sft)
    return _permute(v, lane & 0)


@functools.partial(
    pl.kernel,
    out_type=(jax.ShapeDtypeStruct((NC, NPAD, HC), jnp.float32),
              jax.ShapeDtypeStruct((NC, DPK, HC), jnp.float32)),
    mesh=_MESH,
    scratch_types=[
        pltpu.VMEM((PHE,), jnp.int32),
        pltpu.VMEM((PHE,), jnp.int32),
        pltpu.VMEM((2, BE), jnp.int32),
        pltpu.VMEM((2, BE), jnp.int32),
        pltpu.VMEM((3, BE, HC), jnp.float32),
        pltpu.VMEM((3, BE, HC), jnp.float32),
        pltpu.VMEM((3, BE, HC), jnp.float32),
        pltpu.VMEM((2, BE, HC), jnp.float32),
        pltpu.VMEM((2, BE, HC), jnp.float32),
        pltpu.VMEM((2, BE, HC), jnp.float32),
        pltpu.VMEM((ZROWS, HC), jnp.float32),
        pltpu.VMEM_SHARED((NPAD, HC), jnp.float32),
        pltpu.VMEM_SHARED((DPK, HC), jnp.float32),
        pltpu.SemaphoreType.DMA,
        pltpu.SemaphoreType.DMA,
        pltpu.SemaphoreType.DMA,
        pltpu.SemaphoreType.DMA,
        pltpu.SemaphoreType.DMA,
        pltpu.SemaphoreType.DMA,
    ],
)
def _edge_pass(src_hbm, dst_hbm, q_hbm, k_hbm, v_hbm, ep_hbm, outv, outd,
               sidx, didx, dstb2, d8b2, qb2, kb2, vb2, eb2, ob2, od2,
               zb, accv, dpk, sem_q, sem_k, sem_v, sem_e, sem_ov, sem_od):
    c = lax.axis_index("c")
    s = lax.axis_index("s")
    zero16 = jnp.zeros((16,), jnp.float32)
    lane = lax.iota(jnp.int32, 16)
    one = jnp.int32(1)
    hm = [(one - jnp.minimum(jnp.abs(lane - h), one)).astype(jnp.float32)
          for h in range(NH)]

    def _zrow(i, carry):
        for r in range(HC // 16):
            zb[i, pl.ds(16 * r, 16)] = zero16
        return carry

    lax.fori_loop(0, ZROWS, _zrow, 0)
    for t in range(RPS // ZROWS):
        pltpu.sync_copy(zb, accv.at[pl.ds(s * RPS + t * ZROWS, ZROWS)])
    for t in range(DPS // ZROWS):
        pltpu.sync_copy(zb, dpk.at[pl.ds(s * DPS + t * ZROWS, ZROWS)])
    plsc.subcore_barrier()

    base = (c * NS + s) * EPW

    def _load_idx(ph):
        e0 = base + ph * PHE
        pltpu.sync_copy(src_hbm.at[pl.ds(e0, PHE)], sidx)
        pltpu.sync_copy(dst_hbm.at[pl.ds(e0, PHE)], didx)

    def _issue_qkv(i, g):
        pltpu.async_copy(q_hbm.at[didx.at[pl.ds(BE * i, BE)]], qb2.at[g],
                         sem_q)
        pltpu.async_copy(k_hbm.at[sidx.at[pl.ds(BE * i, BE)]], kb2.at[g],
                         sem_k)
        pltpu.async_copy(v_hbm.at[sidx.at[pl.ds(BE * i, BE)]], vb2.at[g],
                         sem_v)

    def _issue_ep(ph, i, p):
        e0 = base + ph * PHE + i * BE
        pltpu.async_copy(ep_hbm.at[pl.ds(e0, BE)], eb2.at[p], sem_e)

    def _drain_qkv(g):
        pltpu.make_async_copy(ep_hbm.at[pl.ds(0, BE)], qb2.at[g],
                              sem_q).wait()
        pltpu.make_async_copy(ep_hbm.at[pl.ds(0, BE)], kb2.at[g],
                              sem_k).wait()
        pltpu.make_async_copy(ep_hbm.at[pl.ds(0, BE)], vb2.at[g],
                              sem_v).wait()

    def _drain_ep(p):
        pltpu.make_async_copy(ep_hbm.at[pl.ds(0, BE)], eb2.at[p],
                              sem_e).wait()

    def _drain_scatters(p):
        pltpu.make_async_copy(ob2.at[p], accv.at[pl.ds(0, BE)],
                              sem_ov).wait()
        pltpu.make_async_copy(od2.at[p], dpk.at[pl.ds(0, BE)],
                              sem_od).wait()

    def _phase(ph, carry):
        _load_idx(ph)
        _issue_qkv(0, 0)
        _issue_qkv(1, 1)
        _issue_ep(ph, 0, 0)

        def _chunk(i, icarry):
            p = i & 1
            g3 = i - (i // 3) * 3

            @pl.when(i >= 2)
            def _():
                _drain_scatters(p)

            dstv = didx[pl.ds(BE * i, 16)]
            dstb2[p, pl.ds(0, 16)] = dstv
            d8b2[p, pl.ds(0, 16)] = lax.shift_right_logical(dstv, 3)

            @pl.when(i + 2 < PCH)
            def _():
                _issue_qkv(i + 2, (i + 2) - ((i + 2) // 3) * 3)

            @pl.when(i + 1 < PCH)
            def _():
                _issue_ep(ph, i + 1, 1 - p)

            _drain_qkv(g3)
            _drain_ep(p)

            dmod = dstv & 7
            for j in range(BE):
                ep = [eb2[p, j, pl.ds(16 * r, 16)] for r in range(8)]
                pr = [qb2[g3, j, pl.ds(16 * r, 16)] *
                      (kb2[g3, j, pl.ds(16 * r, 16)] + ep[r])
                      for r in range(8)]
                ws = []
                for h in range(NH):
                    a = _lane_sum_splat(pr[2 * h] + pr[2 * h + 1], lane)
                    ws.append(jnp.exp(a * INV_SQRT_C))
                for r in range(8):
                    ob2[p, j, pl.ds(16 * r, 16)] = ws[r // 2] * (
                        vb2[g3, j, pl.ds(16 * r, 16)] + ep[r])
                dv = ws[0] * hm[0]
                for h in range(1, NH):
                    dv = dv + ws[h] * hm[h]
                dsp = _permute(dmod, (lane & 0) + j)
                for pp in range(8):
                    mp = (one - jnp.minimum(dsp ^ pp, one)).astype(
                        jnp.float32)
                    od2[p, j, pl.ds(16 * pp, 16)] = dv * mp

            pltpu.async_copy(ob2.at[p], accv.at[dstb2.at[p]], sem_ov,
                             add=True)
            pltpu.async_copy(od2.at[p], dpk.at[d8b2.at[p]], sem_od,
                             add=True)
            return icarry

        lax.fori_loop(0, PCH, _chunk, 0)
        _drain_scatters(0)
        _drain_scatters(1)
        return carry

    lax.fori_loop(0, NPH, _phase, 0)
    plsc.subcore_barrier()

    pltpu.sync_copy(accv.at[pl.ds(s * RPS, RPS)],
                    outv.at[c, pl.ds(s * RPS, RPS)])
    pltpu.sync_copy(dpk.at[pl.ds(s * DPS, DPS)],
                    outd.at[c, pl.ds(s * DPS, DPS)])


def kernel(x, edge_index, edge_attr, Wq1, bq1, Wk1, bk1, Wv1, bv1, We1, Ws1,
           bs1, Wq2, bq2, Wk2, bk2, Wv2, bv2, We2, Ws2, bs2):
    src = edge_index[0]
    dst = edge_index[1]
    w1 = jnp.concatenate([Wq1, Wk1, Wv1, Ws1], axis=1)
    b1 = jnp.concatenate([bq1, bk1, bv1, bs1]).reshape(1, 4 * HC)
    w2 = jnp.concatenate([Wq2, Wk2, Wv2, Ws2], axis=1)
    b2 = jnp.concatenate([bq2, bk2, bv2, bs2]).reshape(1, 4 * HC)
    wee = jnp.concatenate([We1, We2], axis=1)
    g = jnp.asarray(_G)

    q1, k1, v1, s1 = _node_proj(x, w1, b1)
    ep1, ep2 = _edge_proj(edge_attr, wee)

    av1, ad1 = _edge_pass(src, dst, q1, k1, v1, ep1)
    db1 = _den_proj(ad1, g).reshape(NPAD, HC)
    q2, k2, v2, s2 = _layer2_proj(av1, db1, s1, w2, b2)

    av2, ad2 = _edge_pass(src, dst, q2, k2, v2, ep2)
    db2 = _den_proj(ad2, g).reshape(NPAD, HC)
    return _final(av2, db2, s2)


# speedup vs baseline: 1.7120x; 1.7120x over previous
import functools
import math

import jax
import jax.numpy as jnp
import numpy as np
from jax import lax
from jax.experimental import pallas as pl
from jax.experimental.pallas import tpu as pltpu
from jax.experimental.pallas import tpu_sc as plsc

N = 10000
E = 320000
D = 128
EDIM = 16
NH = 4
CH = 32
HC = NH * CH

NC, NS = 2, 16
NWORK = NC * NS
EPW = E // NWORK
BE = 16
NPH = 5
PHE = EPW // NPH
PCH = PHE // BE
NPAD = 10240
RPS = NPAD // NS
DPK = NPAD // 8
DPS = DPK // NS
ZROWS = 32
INV_SQRT_C = 1.0 / math.sqrt(CH)

NBLK = 5
BN = N // NBLK
EBLK = 64
BEP = E // EBLK
BD = DPK // NBLK

_G = np.zeros((HC, 8 * HC), np.float32)
for _p in range(8):
    for _h in range(NH):
        _G[16 * _p + _h, 128 * _p + 32 * _h:128 * _p + 32 * (_h + 1)] = 1.0


def _dot(a, b):
    return lax.dot_general(a, b, (((1,), (0,)), ((), ())),
                           preferred_element_type=jnp.float32,
                           precision=lax.Precision.HIGHEST)


def _proj_body(x_ref, w_ref, b_ref, oq, ok, ov, os_):
    acc = _dot(x_ref[...], w_ref[...]) + b_ref[...]
    oq[...] = acc[:, 0 * HC:1 * HC]
    ok[...] = acc[:, 1 * HC:2 * HC]
    ov[...] = acc[:, 2 * HC:3 * HC]
    os_[...] = acc[:, 3 * HC:4 * HC]


def _node_proj(x, w, b):
    return pl.pallas_call(
        _proj_body,
        grid=(NBLK,),
        in_specs=[
            pl.BlockSpec((BN, D), lambda i: (i, 0)),
            pl.BlockSpec((D, 4 * HC), lambda i: (0, 0)),
            pl.BlockSpec((1, 4 * HC), lambda i: (0, 0)),
        ],
        out_specs=[pl.BlockSpec((BN, HC), lambda i: (i, 0))] * 4,
        out_shape=[jax.ShapeDtypeStruct((N, HC), jnp.float32)] * 4,
    )(x, w, b)


def _edge_proj_body(ea_ref, w_ref, o1, o2):
    acc = _dot(ea_ref[...], w_ref[...])
    o1[...] = acc[:, 0:HC]
    o2[...] = acc[:, HC:2 * HC]


def _edge_proj(ea, wee):
    return pl.pallas_call(
        _edge_proj_body,
        grid=(EBLK,),
        in_specs=[
            pl.BlockSpec((BEP, EDIM), lambda i: (i, 0)),
            pl.BlockSpec((EDIM, 2 * HC), lambda i: (0, 0)),
        ],
        out_specs=[pl.BlockSpec((BEP, HC), lambda i: (i, 0))] * 2,
        out_shape=[jax.ShapeDtypeStruct((E, HC), jnp.float32)] * 2,
    )(ea, wee)


def _den_proj_body(d0_ref, d1_ref, g_ref, o_ref):
    o_ref[...] = _dot(d0_ref[0] + d1_ref[0], g_ref[...])


def _den_proj(ad, g):
    return pl.pallas_call(
        _den_proj_body,
        grid=(NBLK,),
        in_specs=[
            pl.BlockSpec((1, BD, HC), lambda i: (0, i, 0)),
            pl.BlockSpec((1, BD, HC), lambda i: (1, i, 0)),
            pl.BlockSpec((HC, 8 * HC), lambda i: (0, 0)),
        ],
        out_specs=pl.BlockSpec((BD, 8 * HC), lambda i: (i, 0)),
        out_shape=jax.ShapeDtypeStruct((DPK, 8 * HC), jnp.float32),
    )(ad, ad, g)


def _layer2_body(v0, v1, db, s1_ref, w_ref, b_ref, oq, ok, ov, os_):
    h = jnp.maximum(
        (v0[0] + v1[0]) / (db[...] + 1e-30) + s1_ref[...], 0.0)
    acc = _dot(h, w_ref[...]) + b_ref[...]
    oq[...] = acc[:, 0 * HC:1 * HC]
    ok[...] = acc[:, 1 * HC:2 * HC]
    ov[...] = acc[:, 2 * HC:3 * HC]
    os_[...] = acc[:, 3 * HC:4 * HC]


def _layer2_proj(av, db, s1, w, b):
    return pl.pallas_call(
        _layer2_body,
        grid=(NBLK,),
        in_specs=[
            pl.BlockSpec((1, BN, HC), lambda i: (0, i, 0)),
            pl.BlockSpec((1, BN, HC), lambda i: (1, i, 0)),
            pl.BlockSpec((BN, HC), lambda i: (i, 0)),
            pl.BlockSpec((BN, HC), lambda i: (i, 0)),
            pl.BlockSpec((D, 4 * HC), lambda i: (0, 0)),
            pl.BlockSpec((1, 4 * HC), lambda i: (0, 0)),
        ],
        out_specs=[pl.BlockSpec((BN, HC), lambda i: (i, 0))] * 4,
        out_shape=[jax.ShapeDtypeStruct((N, HC), jnp.float32)] * 4,
    )(av, av, db, s1, w, b)


def _final_body(v0, v1, db, s2_ref, o_ref):
    o_ref[...] = (v0[0] + v1[0]) / (db[...] + 1e-30) + s2_ref[...]


def _final(av, db, s2):
    return pl.pallas_call(
        _final_body,
        grid=(NBLK,),
        in_specs=[
            pl.BlockSpec((1, BN, HC), lambda i: (0, i, 0)),
            pl.BlockSpec((1, BN, HC), lambda i: (1, i, 0)),
            pl.BlockSpec((BN, HC), lambda i: (i, 0)),
            pl.BlockSpec((BN, HC), lambda i: (i, 0)),
        ],
        out_specs=pl.BlockSpec((BN, HC), lambda i: (i, 0)),
        out_shape=jax.ShapeDtypeStruct((N, HC), jnp.float32),
    )(av, av, db, s2)


_MESH = plsc.VectorSubcoreMesh(core_axis_name="c", subcore_axis_name="s",
                               num_cores=NC, num_subcores=NS)

_GDN = lax.GatherDimensionNumbers(offset_dims=(), collapsed_slice_dims=(0,),
                                  start_index_map=(0,))


def _permute(v, idx):
    return lax.gather(v, idx[:, None], _GDN, (1,), indices_are_sorted=False,
                      unique_indices=False,
                      mode=lax.GatherScatterMode.PROMISE_IN_BOUNDS)


def _lane_sum_splat(v, lane):
    for sft in (8, 4, 2, 1):
        v = v + _permute(v, (lane & (sft - 1)) + sft)
    return _permute(v, lane & 0)


@functools.partial(
    pl.kernel,
    out_type=(jax.ShapeDtypeStruct((NC, NPAD, HC), jnp.float32),
              jax.ShapeDtypeStruct((NC, DPK, HC), jnp.float32)),
    mesh=_MESH,
    scratch_types=[
        pltpu.VMEM((PHE,), jnp.int32),
        pltpu.VMEM((PHE,), jnp.int32),
        pltpu.VMEM((4, BE), jnp.int32),
        pltpu.VMEM((4, BE), jnp.int32),
        pltpu.VMEM((4, BE), jnp.int32),
        pltpu.VMEM((2, BE, HC), jnp.float32),
        pltpu.VMEM((2, BE, HC), jnp.float32),
        pltpu.VMEM((2, BE, HC), jnp.float32),
        pltpu.VMEM((2, BE, HC), jnp.float32),
        pltpu.VMEM((2, BE, HC), jnp.float32),
        pltpu.VMEM((2, BE, HC), jnp.float32),
        pltpu.VMEM((ZROWS, HC), jnp.float32),
        pltpu.VMEM_SHARED((NPAD, HC), jnp.float32),
        pltpu.VMEM_SHARED((DPK, HC), jnp.float32),
        pltpu.SemaphoreType.DMA,
        pltpu.SemaphoreType.DMA,
        pltpu.SemaphoreType.DMA,
        pltpu.SemaphoreType.DMA,
        pltpu.SemaphoreType.DMA,
        pltpu.SemaphoreType.DMA,
    ],
)
def _edge_pass(src_hbm, dst_hbm, q_hbm, k_hbm, v_hbm, ep_hbm, outv, outd,
               sidx, didx, srcb2, dstb2, d8b2, qb2, kb2, vb2, eb2, ob2, od2,
               zb, accv, dpk, sem_q, sem_k, sem_v, sem_e, sem_ov, sem_od):
    c = lax.axis_index("c")
    s = lax.axis_index("s")
    zero16 = jnp.zeros((16,), jnp.float32)
    lane = lax.iota(jnp.int32, 16)
    one = jnp.int32(1)
    hm = [(one - jnp.minimum(jnp.abs(lane - h), one)).astype(jnp.float32)
          for h in range(NH)]

    def _zrow(i, carry):
        for r in range(HC // 16):
            zb[i, pl.ds(16 * r, 16)] = zero16
        return carry

    lax.fori_loop(0, ZROWS, _zrow, 0)
    for t in range(RPS // ZROWS):
        pltpu.sync_copy(zb, accv.at[pl.ds(s * RPS + t * ZROWS, ZROWS)])
    for t in range(DPS // ZROWS):
        pltpu.sync_copy(zb, dpk.at[pl.ds(s * DPS + t * ZROWS, ZROWS)])
    pltpu.sync_copy(zb.at[pl.ds(0, DPS % ZROWS)],
                    dpk.at[pl.ds(s * DPS + (DPS // ZROWS) * ZROWS,
                                 DPS % ZROWS)])
    plsc.subcore_barrier()

    base = (c * NS + s) * EPW

    def _load_idx(ph):
        e0 = base + ph * PHE
        pltpu.sync_copy(src_hbm.at[pl.ds(e0, PHE)], sidx)
        pltpu.sync_copy(dst_hbm.at[pl.ds(e0, PHE)], didx)

    def _stage(i, sl):
        srcv = sidx[pl.ds(BE * i, 16)]
        dstv = didx[pl.ds(BE * i, 16)]
        srcb2[sl, pl.ds(0, 16)] = srcv
        dstb2[sl, pl.ds(0, 16)] = dstv
        d8b2[sl, pl.ds(0, 16)] = lax.shift_right_logical(dstv, 3)

    def _issue_gathers(ph, i, p, sl):
        pltpu.async_copy(q_hbm.at[dstb2.at[sl]], qb2.at[p], sem_q)
        pltpu.async_copy(k_hbm.at[srcb2.at[sl]], kb2.at[p], sem_k)
        pltpu.async_copy(v_hbm.at[srcb2.at[sl]], vb2.at[p], sem_v)
        e0 = base + ph * PHE + i * BE
        pltpu.async_copy(ep_hbm.at[pl.ds(e0, BE)], eb2.at[p], sem_e)

    def _drain_scatters(p):
        pltpu.make_async_copy(ob2.at[p], accv.at[pl.ds(0, BE)],
                              sem_ov).wait()
        pltpu.make_async_copy(od2.at[p], dpk.at[pl.ds(0, BE)],
                              sem_od).wait()

    def _drain_gathers(p):
        pltpu.make_async_copy(ep_hbm.at[pl.ds(0, BE)], qb2.at[p],
                              sem_q).wait()
        pltpu.make_async_copy(ep_hbm.at[pl.ds(0, BE)], kb2.at[p],
                              sem_k).wait()
        pltpu.make_async_copy(ep_hbm.at[pl.ds(0, BE)], vb2.at[p],
                              sem_v).wait()
        pltpu.make_async_copy(ep_hbm.at[pl.ds(0, BE)], eb2.at[p],
                              sem_e).wait()

    def _phase(ph, carry):
        _load_idx(ph)
        _stage(0, 0)
        _stage(1, 1)
        lax.fori_loop(0, 4, _zrow, 0)
        _issue_gathers(ph, 0, 0, 0)

        def _chunk(i, icarry):
            p = i & 1
            sl = i & 3

            @pl.when(i >= 2)
            def _():
                _drain_scatters(p)

            _drain_gathers(p)

            @pl.when(i + 2 < PCH)
            def _():
                _stage(i + 2, (i + 2) & 3)

            @pl.when(i + 1 < PCH)
            def _():
                _issue_gathers(ph, i + 1, 1 - p, (i + 1) & 3)

            dstv = dstb2[sl, pl.ds(0, 16)]
            dmod = dstv & 7
            for j in range(BE):
                ep = [eb2[p, j, pl.ds(16 * r, 16)] for r in range(8)]
                pr = [qb2[p, j, pl.ds(16 * r, 16)] *
                      (kb2[p, j, pl.ds(16 * r, 16)] + ep[r])
                      for r in range(8)]
                ws = []
                for h in range(NH):
                    a = _lane_sum_splat(pr[2 * h] + pr[2 * h + 1], lane)
                    ws.append(jnp.exp(a * INV_SQRT_C))
                for r in range(8):
                    ob2[p, j, pl.ds(16 * r, 16)] = ws[r // 2] * (
                        vb2[p, j, pl.ds(16 * r, 16)] + ep[r])
                dv = ws[0] * hm[0]
                for h in range(1, NH):
                    dv = dv + ws[h] * hm[h]
                dsp = _permute(dmod, (lane & 0) + j)
                for pp in range(8):
                    mp = (one - jnp.minimum(dsp ^ pp, one)).astype(
                        jnp.float32)
                    od2[p, j, pl.ds(16 * pp, 16)] = dv * mp

            pltpu.async_copy(ob2.at[p], accv.at[dstb2.at[sl]], sem_ov,
                             add=True)
            pltpu.async_copy(od2.at[p], dpk.at[d8b2.at[sl]], sem_od,
                             add=True)
            return icarry

        lax.fori_loop(0, PCH, _chunk, 0)
        _drain_scatters(0)
        _drain_scatters(1)
        return carry

    lax.fori_loop(0, NPH, _phase, 0)
    plsc.subcore_barrier()

    pltpu.sync_copy(accv.at[pl.ds(s * RPS, RPS)],
                    outv.at[c, pl.ds(s * RPS, RPS)])
    pltpu.sync_copy(dpk.at[pl.ds(s * DPS, DPS)],
                    outd.at[c, pl.ds(s * DPS, DPS)])


def kernel(x, edge_index, edge_attr, Wq1, bq1, Wk1, bk1, Wv1, bv1, We1, Ws1,
           bs1, Wq2, bq2, Wk2, bk2, Wv2, bv2, We2, Ws2, bs2):
    src = edge_index[0]
    dst = edge_index[1]
    w1 = jnp.concatenate([Wq1, Wk1, Wv1, Ws1], axis=1)
    b1 = jnp.concatenate([bq1, bk1, bv1, bs1]).reshape(1, 4 * HC)
    w2 = jnp.concatenate([Wq2, Wk2, Wv2, Ws2], axis=1)
    b2 = jnp.concatenate([bq2, bk2, bv2, bs2]).reshape(1, 4 * HC)
    wee = jnp.concatenate([We1, We2], axis=1)
    g = jnp.asarray(_G)

    q1, k1, v1, s1 = _node_proj(x, w1, b1)
    ep1, ep2 = _edge_proj(edge_attr, wee)

    av1, ad1 = _edge_pass(src, dst, q1, k1, v1, ep1)
    db1 = _den_proj(ad1, g).reshape(NPAD, HC)
    q2, k2, v2, s2 = _layer2_proj(av1, db1, s1, w2, b2)

    av2, ad2 = _edge_pass(src, dst, q2, k2, v2, ep2)
    db2 = _den_proj(ad2, g).reshape(NPAD, HC)
    return _final(av2, db2, s2)
